# baseline (device time: 107730 ns/iter reference)
import jax
import jax.numpy as jnp
from jax import lax
from jax.experimental import pallas as pl
from jax.experimental.pallas import tpu as pltpu

T = 4096
D = 1024
BLK = 512
NBLK = T // BLK
REM_BITS = (8, 7, 6, 5, 4)
NSEM = NBLK + len(REM_BITS)


def _pack_exchange(L_arr, order_col, x_bf):

    def body(L_ref, pos_ref, x_ref, out_ref, send_ref, recv_ref,
             send_sems, recv_sems):
        my_x = lax.axis_index("x")
        my_y = lax.axis_index("y")
        my_z = lax.axis_index("z")
        partner = (1 - my_x, my_y, my_z)

        barrier = pltpu.get_barrier_semaphore()
        pl.semaphore_signal(
            barrier, inc=1, device_id=partner,
            device_id_type=pl.DeviceIdType.MESH,
        )
        pl.semaphore_wait(barrier, 1)

        L = L_ref[0]
        K = T - L
        C = (L + 15) & ~15
        n_full = C >> 9
        rem_base = n_full << 9

        def chunk(off, size, si):
            return pltpu.make_async_remote_copy(
                src_ref=send_ref.at[pl.ds(off, size)],
                dst_ref=recv_ref.at[pl.ds(off, size)],
                send_sem=send_sems.at[si],
                recv_sem=recv_sems.at[si],
                device_id=partner,
                device_id_type=pl.DeviceIdType.MESH,
            )

        def rem_chunk(b, si):
            off = pl.multiple_of(
                rem_base + (((C & 511) >> (b + 1)) << (b + 1)), 16
            )
            return chunk(off, 1 << b, si)

        rowid = lax.broadcasted_iota(jnp.int32, (BLK, T), 0)
        for c in range(NBLK):
            p = (pos_ref[...] == rowid + c * BLK).astype(jnp.bfloat16)
            send_ref[pl.ds(c * BLK, BLK), :] = jnp.dot(
                p, x_ref[...], preferred_element_type=jnp.float32
            ).astype(jnp.bfloat16)

            @pl.when(c < n_full)
            def _(c=c):
                chunk(c * BLK, BLK, c).start()

            for bi, b in enumerate(REM_BITS):
                @pl.when((c == n_full) & ((((C & 511) >> b) & 1) == 1))
                def _(b=b, bi=bi):
                    rem_chunk(b, NBLK + bi).start()

        QB = T // 4

        @pl.when(my_x == 0)
        def _():
            rolled_s = pltpu.roll(send_ref[...], -L, 0)
            for h in range(4):
                out_ref[pl.ds(h * QB, QB), :] = (
                    rolled_s[h * QB:(h + 1) * QB].astype(jnp.float32))

        @pl.when(my_x == 1)
        def _():
            for h in range(4):
                out_ref[pl.ds(h * QB, QB), :] = (
                    send_ref[pl.ds(h * QB, QB), :].astype(jnp.float32))

        for c in range(NBLK):
            @pl.when(c < n_full)
            def _(c=c):
                chunk(c * BLK, BLK, c).wait_recv()

        for bi, b in enumerate(REM_BITS):
            @pl.when((((C & 511) >> b) & 1) == 1)
            def _(b=b, bi=bi):
                rem_chunk(b, NBLK + bi).wait_recv()

        @pl.when(my_x == 0)
        def _():
            rolled_r = pltpu.roll(recv_ref[...], K, 0)
            for h in range(4):
                sl = pl.ds(h * QB, QB)
                rows = lax.broadcasted_iota(jnp.int32, (QB, 1), 0) + h * QB
                out_ref[sl, :] = jnp.where(
                    rows < K,
                    out_ref[sl, :],
                    rolled_r[h * QB:(h + 1) * QB].astype(jnp.float32),
                )

        @pl.when(my_x == 1)
        def _():
            for h in range(4):
                sl = pl.ds(h * QB, QB)
                rows = lax.broadcasted_iota(jnp.int32, (QB, 1), 0) + h * QB
                out_ref[sl, :] = jnp.where(
                    rows < L,
                    recv_ref[sl, :].astype(jnp.float32),
                    out_ref[sl, :],
                )

        for c in range(NBLK):
            @pl.when(c < n_full)
            def _(c=c):
                chunk(c * BLK, BLK, c).wait_send()

        for bi, b in enumerate(REM_BITS):
            @pl.when((((C & 511) >> b) & 1) == 1)
            def _(b=b, bi=bi):
                rem_chunk(b, NBLK + bi).wait_send()

    return pl.pallas_call(
        body,
        out_shape=jax.ShapeDtypeStruct((T, D), jnp.float32),
        in_specs=[
            pl.BlockSpec(memory_space=pltpu.SMEM),
            pl.BlockSpec(memory_space=pltpu.VMEM),
            pl.BlockSpec(memory_space=pltpu.VMEM),
        ],
        out_specs=pl.BlockSpec(memory_space=pltpu.VMEM),
        scratch_shapes=[
            pltpu.VMEM((T, D), jnp.bfloat16),
            pltpu.VMEM((T, D), jnp.bfloat16),
            pltpu.SemaphoreType.DMA((NSEM,)),
            pltpu.SemaphoreType.DMA((NSEM,)),
        ],
        compiler_params=pltpu.CompilerParams(
            collective_id=0, vmem_limit_bytes=100 * 1024 * 1024
        ),
    )(L_arr, order_col, x_bf)


def kernel(x, dest):
    my_x = lax.axis_index("x")

    send_mask = dest != my_x
    cs = jnp.cumsum(send_mask.astype(jnp.int32), dtype=jnp.int32)
    L = cs[T - 1]
    j = jnp.arange(T, dtype=jnp.int32)
    pos = jnp.where(send_mask, cs - 1, L + j - cs)

    return _pack_exchange(
        L.reshape((1,)),
        pos.reshape(1, T),
        x.astype(jnp.bfloat16),
    )


# device time: 96736 ns/iter; 1.1136x vs baseline; 1.1136x over previous
import jax
import jax.numpy as jnp
from jax import lax
from jax.experimental import pallas as pl
from jax.experimental.pallas import tpu as pltpu

T = 4096
D = 1024
BLK = 512
NBLK = T // BLK
REM_BITS = (8, 7, 6, 5, 4)
NSEM = NBLK + len(REM_BITS)


def _pack_exchange(L_arr, order_col, x_bf):

    def body(L_ref, pos_ref, x_ref, out_ref, send_ref, recv_ref,
             send_sems, recv_sems):
        my_x = lax.axis_index("x")
        my_y = lax.axis_index("y")
        my_z = lax.axis_index("z")
        partner = (1 - my_x, my_y, my_z)

        barrier = pltpu.get_barrier_semaphore()
        pl.semaphore_signal(
            barrier, inc=1, device_id=partner,
            device_id_type=pl.DeviceIdType.MESH,
        )
        pl.semaphore_wait(barrier, 1)

        L = L_ref[0]
        K = T - L
        C = (L + 15) & ~15
        n_full = C >> 9
        rem_base = n_full << 9

        def chunk(off, size, si):
            return pltpu.make_async_remote_copy(
                src_ref=send_ref.at[pl.ds(off, size)],
                dst_ref=recv_ref.at[pl.ds(off, size)],
                send_sem=send_sems.at[si],
                recv_sem=recv_sems.at[si],
                device_id=partner,
                device_id_type=pl.DeviceIdType.MESH,
            )

        def rem_chunk(b, si):
            off = pl.multiple_of(
                rem_base + (((C & 511) >> (b + 1)) << (b + 1)), 16
            )
            return chunk(off, 1 << b, si)

        rowid = lax.broadcasted_iota(jnp.int32, (BLK, T), 0)
        for c in range(NBLK):
            p = (pos_ref[...] == rowid + c * BLK).astype(jnp.bfloat16)
            send_ref[pl.ds(c * BLK, BLK), :] = jnp.dot(
                p, x_ref[...], preferred_element_type=jnp.float32
            ).astype(jnp.bfloat16)

            @pl.when(c < n_full)
            def _(c=c):
                chunk(c * BLK, BLK, c).start()

            for bi, b in enumerate(REM_BITS):
                @pl.when((c == n_full) & ((((C & 511) >> b) & 1) == 1))
                def _(b=b, bi=bi):
                    rem_chunk(b, NBLK + bi).start()

        rows = lax.broadcasted_iota(jnp.int32, (T, 1), 0)

        @pl.when(my_x == 0)
        def _():
            out_ref[...] = pltpu.roll(send_ref[...], -L, 0)

        @pl.when(my_x == 1)
        def _():
            out_ref[...] = send_ref[...]

        for c in range(NBLK):
            @pl.when(c < n_full)
            def _(c=c):
                chunk(c * BLK, BLK, c).wait_recv()

        for bi, b in enumerate(REM_BITS):
            @pl.when((((C & 511) >> b) & 1) == 1)
            def _(b=b, bi=bi):
                rem_chunk(b, NBLK + bi).wait_recv()

        @pl.when(my_x == 0)
        def _():
            recv = pltpu.roll(recv_ref[...], K, 0)
            out_ref[...] = jnp.where(rows < K, out_ref[...], recv)

        @pl.when(my_x == 1)
        def _():
            out_ref[...] = jnp.where(rows < L, recv_ref[...], out_ref[...])

        for c in range(NBLK):
            @pl.when(c < n_full)
            def _(c=c):
                chunk(c * BLK, BLK, c).wait_send()

        for bi, b in enumerate(REM_BITS):
            @pl.when((((C & 511) >> b) & 1) == 1)
            def _(b=b, bi=bi):
                rem_chunk(b, NBLK + bi).wait_send()

    return pl.pallas_call(
        body,
        out_shape=jax.ShapeDtypeStruct((T, D), jnp.bfloat16),
        in_specs=[
            pl.BlockSpec(memory_space=pltpu.SMEM),
            pl.BlockSpec(memory_space=pltpu.VMEM),
            pl.BlockSpec(memory_space=pltpu.VMEM),
        ],
        out_specs=pl.BlockSpec(memory_space=pltpu.VMEM),
        scratch_shapes=[
            pltpu.VMEM((T, D), jnp.bfloat16),
            pltpu.VMEM((T, D), jnp.bfloat16),
            pltpu.SemaphoreType.DMA((NSEM,)),
            pltpu.SemaphoreType.DMA((NSEM,)),
        ],
        compiler_params=pltpu.CompilerParams(
            collective_id=0, vmem_limit_bytes=100 * 1024 * 1024
        ),
    )(L_arr, order_col, x_bf)


def _cast_f32(x_bf):

    def body(x_ref, o_ref):
        o_ref[...] = x_ref[...].astype(jnp.float32)

    return pl.pallas_call(
        body,
        grid=(NBLK,),
        in_specs=[pl.BlockSpec((BLK, D), lambda c: (c, 0))],
        out_specs=pl.BlockSpec((BLK, D), lambda c: (c, 0)),
        out_shape=jax.ShapeDtypeStruct((T, D), jnp.float32),
    )(x_bf)


def kernel(x, dest):
    my_x = lax.axis_index("x")

    send_mask = dest != my_x
    cs = jnp.cumsum(send_mask.astype(jnp.int32), dtype=jnp.int32)
    L = cs[T - 1]
    j = jnp.arange(T, dtype=jnp.int32)
    pos = jnp.where(send_mask, cs - 1, L + j - cs)

    out_bf = _pack_exchange(
        L.reshape((1,)),
        pos.reshape(1, T),
        x.astype(jnp.bfloat16),
    )
    return _cast_f32(out_bf)


# device time: 92444 ns/iter; 1.1654x vs baseline; 1.0464x over previous
import jax
import jax.numpy as jnp
from jax import lax
from jax.experimental import pallas as pl
from jax.experimental.pallas import tpu as pltpu

T = 4096
D = 1024
BLK = 256
BLK_LOG2 = 8
NBLK = T // BLK
REM_BITS = (7, 6, 5, 4)
NSEM = NBLK + len(REM_BITS)
QB = 1024


def _pack_exchange(L_arr, order_col, x_bf):

    def body(L_ref, pos_ref, x_ref, out_ref, send_ref, recv_ref,
             send_sems, recv_sems):
        my_x = lax.axis_index("x")
        my_y = lax.axis_index("y")
        my_z = lax.axis_index("z")
        partner = (1 - my_x, my_y, my_z)

        barrier = pltpu.get_barrier_semaphore()
        pl.semaphore_signal(
            barrier, inc=1, device_id=partner,
            device_id_type=pl.DeviceIdType.MESH,
        )
        pl.semaphore_wait(barrier, 1)

        L = L_ref[0]
        K = T - L
        C = (L + 15) & ~15
        n_full = C >> BLK_LOG2
        rem_base = n_full << BLK_LOG2

        def chunk(off, size, si):
            return pltpu.make_async_remote_copy(
                src_ref=send_ref.at[pl.ds(off, size)],
                dst_ref=recv_ref.at[pl.ds(off, size)],
                send_sem=send_sems.at[si],
                recv_sem=recv_sems.at[si],
                device_id=partner,
                device_id_type=pl.DeviceIdType.MESH,
            )

        def rem_chunk(b, si):
            off = pl.multiple_of(
                rem_base + (((C & (BLK - 1)) >> (b + 1)) << (b + 1)), 16
            )
            return chunk(off, 1 << b, si)

        rowid = lax.broadcasted_iota(jnp.int32, (BLK, T), 0)
        for c in range(NBLK):
            p = (pos_ref[...] == rowid + c * BLK).astype(jnp.bfloat16)
            send_ref[pl.ds(c * BLK, BLK), :] = jnp.dot(
                p, x_ref[...], preferred_element_type=jnp.float32
            ).astype(jnp.bfloat16)

            @pl.when(c < n_full)
            def _(c=c):
                chunk(c * BLK, BLK, c).start()

            for bi, b in enumerate(REM_BITS):
                @pl.when((c == n_full) & ((((C & (BLK - 1)) >> b) & 1) == 1))
                def _(b=b, bi=bi):
                    rem_chunk(b, NBLK + bi).start()

        @pl.when(my_x == 0)
        def _():
            out_ref[...] = pltpu.roll(send_ref[...], -L, 0)

        @pl.when(my_x == 1)
        def _():
            out_ref[...] = send_ref[...]

        for c in range(NBLK):
            @pl.when(c < n_full)
            def _(c=c):
                chunk(c * BLK, BLK, c).wait_recv()

        for bi, b in enumerate(REM_BITS):
            @pl.when((((C & (BLK - 1)) >> b) & 1) == 1)
            def _(b=b, bi=bi):
                rem_chunk(b, NBLK + bi).wait_recv()

        @pl.when(my_x == 0)
        def _():
            recv = pltpu.roll(recv_ref[...], K, 0)
            for h in range(T // QB):
                @pl.when(K < (h + 1) * QB)
                def _(h=h):
                    sl = pl.ds(h * QB, QB)
                    rq = lax.broadcasted_iota(jnp.int32, (QB, 1), 0) + h * QB
                    out_ref[sl, :] = jnp.where(
                        rq < K, out_ref[sl, :], recv[h * QB:(h + 1) * QB]
                    )

        @pl.when(my_x == 1)
        def _():
            for h in range(T // QB):
                @pl.when(L > h * QB)
                def _(h=h):
                    sl = pl.ds(h * QB, QB)
                    rq = lax.broadcasted_iota(jnp.int32, (QB, 1), 0) + h * QB
                    out_ref[sl, :] = jnp.where(
                        rq < L, recv_ref[sl, :], out_ref[sl, :]
                    )

        for c in range(NBLK):
            @pl.when(c < n_full)
            def _(c=c):
                chunk(c * BLK, BLK, c).wait_send()

        for bi, b in enumerate(REM_BITS):
            @pl.when((((C & (BLK - 1)) >> b) & 1) == 1)
            def _(b=b, bi=bi):
                rem_chunk(b, NBLK + bi).wait_send()

    return pl.pallas_call(
        body,
        out_shape=jax.ShapeDtypeStruct((T, D), jnp.bfloat16),
        in_specs=[
            pl.BlockSpec(memory_space=pltpu.SMEM),
            pl.BlockSpec(memory_space=pltpu.VMEM),
            pl.BlockSpec(memory_space=pltpu.VMEM),
        ],
        out_specs=pl.BlockSpec(memory_space=pltpu.VMEM),
        scratch_shapes=[
            pltpu.VMEM((T, D), jnp.bfloat16),
            pltpu.VMEM((T, D), jnp.bfloat16),
            pltpu.SemaphoreType.DMA((NSEM,)),
            pltpu.SemaphoreType.DMA((NSEM,)),
        ],
        compiler_params=pltpu.CompilerParams(
            collective_id=0, vmem_limit_bytes=100 * 1024 * 1024
        ),
    )(L_arr, order_col, x_bf)


def _cast_f32(x_bf):

    def body(x_ref, o_ref):
        o_ref[...] = x_ref[...].astype(jnp.float32)

    return pl.pallas_call(
        body,
        grid=(NBLK,),
        in_specs=[pl.BlockSpec((BLK, D), lambda c: (c, 0))],
        out_specs=pl.BlockSpec((BLK, D), lambda c: (c, 0)),
        out_shape=jax.ShapeDtypeStruct((T, D), jnp.float32),
    )(x_bf)


def kernel(x, dest):
    my_x = lax.axis_index("x")

    send_mask = dest != my_x
    cs = jnp.cumsum(send_mask.astype(jnp.int32), dtype=jnp.int32)
    L = cs[T - 1]
    j = jnp.arange(T, dtype=jnp.int32)
    pos = jnp.where(send_mask, cs - 1, L + j - cs)

    out_bf = _pack_exchange(
        L.reshape((1,)),
        pos.reshape(1, T),
        x.astype(jnp.bfloat16),
    )
    return _cast_f32(out_bf)


# device time: 90132 ns/iter; 1.1952x vs baseline; 1.0257x over previous
import jax
import jax.numpy as jnp
from jax import lax
from jax.experimental import pallas as pl
from jax.experimental.pallas import tpu as pltpu

T = 4096
D = 1024
BLK = 256
BLK_LOG2 = 8
NBLK = T // BLK
REM_BITS = (7, 6, 5, 4)
NSEM = NBLK + len(REM_BITS)
QB = 1024


def _pack_exchange(L_arr, order_col, x_bf):

    def body(L_ref, pos_ref, xbf_ref, out_ref, send_ref, recv_ref,
             send_sems, recv_sems):
        my_x = lax.axis_index("x")
        my_y = lax.axis_index("y")
        my_z = lax.axis_index("z")
        partner = (1 - my_x, my_y, my_z)

        barrier = pltpu.get_barrier_semaphore()
        pl.semaphore_signal(
            barrier, inc=1, device_id=partner,
            device_id_type=pl.DeviceIdType.MESH,
        )
        pl.semaphore_wait(barrier, 1)

        L = L_ref[0]
        K = T - L
        C = (L + 15) & ~15
        n_full = C >> BLK_LOG2
        rem_base = n_full << BLK_LOG2

        def chunk(off, size, si):
            return pltpu.make_async_remote_copy(
                src_ref=send_ref.at[pl.ds(off, size)],
                dst_ref=recv_ref.at[pl.ds(off, size)],
                send_sem=send_sems.at[si],
                recv_sem=recv_sems.at[si],
                device_id=partner,
                device_id_type=pl.DeviceIdType.MESH,
            )

        def rem_chunk(b, si):
            off = pl.multiple_of(
                rem_base + (((C & (BLK - 1)) >> (b + 1)) << (b + 1)), 16
            )
            return chunk(off, 1 << b, si)

        rowid = lax.broadcasted_iota(jnp.int32, (BLK, T), 0)
        for c in range(NBLK):
            p = (pos_ref[...] == rowid + c * BLK).astype(jnp.bfloat16)
            send_ref[pl.ds(c * BLK, BLK), :] = jnp.dot(
                p, xbf_ref[...], preferred_element_type=jnp.float32
            ).astype(jnp.bfloat16)

            @pl.when(c < n_full)
            def _(c=c):
                chunk(c * BLK, BLK, c).start()

            for bi, b in enumerate(REM_BITS):
                @pl.when((c == n_full) & ((((C & (BLK - 1)) >> b) & 1) == 1))
                def _(b=b, bi=bi):
                    rem_chunk(b, NBLK + bi).start()

        @pl.when(my_x == 0)
        def _():
            out_ref[...] = pltpu.roll(send_ref[...], -L, 0)

        @pl.when(my_x == 1)
        def _():
            out_ref[...] = send_ref[...]

        for c in range(NBLK):
            @pl.when(c < n_full)
            def _(c=c):
                chunk(c * BLK, BLK, c).wait_recv()

        for bi, b in enumerate(REM_BITS):
            @pl.when((((C & (BLK - 1)) >> b) & 1) == 1)
            def _(b=b, bi=bi):
                rem_chunk(b, NBLK + bi).wait_recv()

        @pl.when(my_x == 0)
        def _():
            recv = pltpu.roll(recv_ref[...], K, 0)
            for h in range(T // QB):
                @pl.when(K < (h + 1) * QB)
                def _(h=h):
                    sl = pl.ds(h * QB, QB)
                    rq = lax.broadcasted_iota(jnp.int32, (QB, 1), 0) + h * QB
                    out_ref[sl, :] = jnp.where(
                        rq < K, out_ref[sl, :], recv[h * QB:(h + 1) * QB]
                    )

        @pl.when(my_x == 1)
        def _():
            for h in range(T // QB):
                @pl.when(L > h * QB)
                def _(h=h):
                    sl = pl.ds(h * QB, QB)
                    rq = lax.broadcasted_iota(jnp.int32, (QB, 1), 0) + h * QB
                    out_ref[sl, :] = jnp.where(
                        rq < L, recv_ref[sl, :], out_ref[sl, :]
                    )

        for c in range(NBLK):
            @pl.when(c < n_full)
            def _(c=c):
                chunk(c * BLK, BLK, c).wait_send()

        for bi, b in enumerate(REM_BITS):
            @pl.when((((C & (BLK - 1)) >> b) & 1) == 1)
            def _(b=b, bi=bi):
                rem_chunk(b, NBLK + bi).wait_send()

    return pl.pallas_call(
        body,
        out_shape=jax.ShapeDtypeStruct((T, D), jnp.bfloat16),
        in_specs=[
            pl.BlockSpec(memory_space=pltpu.SMEM),
            pl.BlockSpec(memory_space=pltpu.VMEM),
            pl.BlockSpec(memory_space=pltpu.VMEM),
        ],
        out_specs=pl.BlockSpec(memory_space=pltpu.VMEM),
        scratch_shapes=[
            pltpu.VMEM((T, D), jnp.bfloat16),
            pltpu.VMEM((T, D), jnp.bfloat16),
            pltpu.SemaphoreType.DMA((NSEM,)),
            pltpu.SemaphoreType.DMA((NSEM,)),
        ],
        compiler_params=pltpu.CompilerParams(
            collective_id=0, vmem_limit_bytes=100 * 1024 * 1024
        ),
    )(L_arr, order_col, x_bf)


def _cast_f32(x_bf):

    def body(x_ref, o_ref):
        o_ref[...] = x_ref[...].astype(jnp.float32)

    CB = 512
    return pl.pallas_call(
        body,
        grid=(T // CB,),
        in_specs=[pl.BlockSpec((CB, D), lambda c: (c, 0))],
        out_specs=pl.BlockSpec((CB, D), lambda c: (c, 0)),
        out_shape=jax.ShapeDtypeStruct((T, D), jnp.float32),
    )(x_bf)


def kernel(x, dest):
    my_x = lax.axis_index("x")

    send_mask = dest != my_x
    cs = jnp.cumsum(send_mask.astype(jnp.int32), dtype=jnp.int32)
    L = cs[T - 1]
    j = jnp.arange(T, dtype=jnp.int32)
    pos = jnp.where(send_mask, cs - 1, L + j - cs)

    out_bf = _pack_exchange(
        L.reshape((1,)), pos.reshape(1, T), x.astype(jnp.bfloat16)
    )
    return _cast_f32(out_bf)
